# speedup vs baseline: 1.2462x; 1.2462x over previous
import functools

import jax
import jax.numpy as jnp
from jax import lax
from jax.experimental import pallas as pl
from jax.experimental.pallas import tpu as pltpu
from jax.experimental.pallas import tpu_sc as plsc

_B = 1024
_D = 64 * 96
_NC = 2
_NS = 16
_NW = _NC * _NS
_BPW = _B // _NW
_CH = 8


def _gather_body(table_hbm, idx_hbm, out_hbm, idx_v, buf0, buf1, sem0, sem1):
    wid = lax.axis_index("s") * _NC + lax.axis_index("c")
    base = wid * _BPW
    pltpu.sync_copy(idx_hbm.at[pl.ds(base, _BPW)], idx_v)

    bufs = (buf0, buf1)
    sems = (sem0, sem1)
    n_chunks = _BPW // _CH

    pltpu.async_copy(table_hbm.at[idx_v.at[pl.ds(0, _CH)]], bufs[0], sems[0])
    for c in range(n_chunks):
        cur = c % 2
        nxt = (c + 1) % 2
        if c + 1 < n_chunks:
            pltpu.async_copy(
                table_hbm.at[idx_v.at[pl.ds((c + 1) * _CH, _CH)]],
                bufs[nxt],
                sems[nxt],
            )
        pltpu.make_async_copy(
            table_hbm.at[idx_v.at[pl.ds(c * _CH, _CH)]], bufs[cur], sems[cur]
        ).wait()
        pltpu.sync_copy(bufs[cur], out_hbm.at[pl.ds(base + c * _CH, _CH)])


@functools.partial(
    pl.kernel,
    mesh=plsc.VectorSubcoreMesh(core_axis_name="c", subcore_axis_name="s"),
    out_type=jax.ShapeDtypeStruct((_B, _D), jnp.float32),
    scratch_types=[
        pltpu.VMEM((_BPW,), jnp.int32),
        pltpu.VMEM((_CH, _D), jnp.float32),
        pltpu.VMEM((_CH, _D), jnp.float32),
        pltpu.SemaphoreType.DMA,
        pltpu.SemaphoreType.DMA,
    ],
)
def _gather(table_hbm, idx_hbm, out_hbm, idx_v, buf0, buf1, sem0, sem1):
    _gather_body(table_hbm, idx_hbm, out_hbm, idx_v, buf0, buf1, sem0, sem1)


def kernel(x, y, audio, pose, dummy):
    idx = jax.random.randint(
        jax.random.key(42), (y.shape[0],), 0, pose.shape[0]
    ).astype(jnp.int32)
    table = pose.reshape(pose.shape[0], _D)
    out = _gather(table, idx)
    return out.reshape(_B, 64, 96), jnp.sum(dummy)
